# Initial kernel scaffold; baseline (speedup 1.0000x reference)
import jax
import jax.numpy as jnp
from jax.experimental import pallas as pl


def kernel(x, edge_index, batch, W_rel0, b_rel0, W_root0, W_res0, b_res0, gamma0, beta0, W_rel1, b_rel1, W_root1, W_res1, b_res1, gamma1, beta1, W_rel2, b_rel2, W_root2, W_res2, b_res2, gamma2, beta2, W_final):
    raise NotImplementedError("write your pallas kernel here")


# speedup vs baseline: 2.6110x; 2.6110x over previous
import functools

import jax
import jax.numpy as jnp
from jax import lax
from jax.experimental import pallas as pl
from jax.experimental.pallas import tpu as pltpu
from jax.experimental.pallas import tpu_sc as plsc

N = 10000
E = 160000
D_IN = 256
D_H = 512
D_OUT = 128
G = 64
EPS = 1e-5

NPAD = 10240
BLK = 256
NBLK = NPAD // BLK
NSUB = 16
NCORE = 2
CW = 64
EB = 79
ET = EB * 128
EPAD = NSUB * ET
ROWS_PER_SUB = NPAD // NSUB


def _make_segsum(num_chunks):
    C = num_chunks
    CPC = C // NCORE

    mesh = plsc.VectorSubcoreMesh(core_axis_name="c", subcore_axis_name="s",
                                  num_cores=NCORE, num_subcores=NSUB)
    out_type = [jax.ShapeDtypeStruct((NPAD, CW), jnp.float32)] * C
    scratch = [
        pltpu.VMEM((EB, 128), jnp.int32),
        pltpu.VMEM((EB, 128), jnp.int32),
        pltpu.VMEM((128, CW), jnp.float32),
        pltpu.VMEM((128, CW), jnp.float32),
        pltpu.VMEM((128, CW), jnp.float32),
        pltpu.VMEM_SHARED((NPAD, CW), jnp.float32),
        pltpu.SemaphoreType.DMA,
    ]

    @functools.partial(pl.kernel, out_type=out_type, mesh=mesh,
                       scratch_types=scratch,
                       compiler_params=pltpu.CompilerParams(
                           use_tc_tiling_on_sc=False))
    def seg(*refs):
        h_refs = refs[:C]
        src_hbm = refs[C]
        dst_hbm = refs[C + 1]
        zeros_hbm = refs[C + 2]
        out_refs = refs[C + 3:C + 3 + C]
        src_v, dst_v, rows_v, zbuf, stage, acc, sem = refs[C + 3 + C:]

        cid = lax.axis_index("c")
        sid = lax.axis_index("s")
        row0 = sid * ROWS_PER_SUB

        pltpu.sync_copy(src_hbm.at[sid], src_v)
        pltpu.sync_copy(dst_hbm.at[sid], dst_v)
        pltpu.sync_copy(zeros_hbm, zbuf)

        for cc in range(NCORE):
            @pl.when(cid == cc)
            def _():
                for q in range(CPC):
                    ch = cc * CPC + q
                    h_hbm = h_refs[ch]
                    out_hbm = out_refs[ch]

                    for k in range(ROWS_PER_SUB // 128):
                        pltpu.sync_copy(
                            zbuf, acc.at[pl.ds(row0 + k * 128, 128)])
                    plsc.subcore_barrier()

                    def body(j, carry):
                        pltpu.async_copy(
                            h_hbm.at[src_v.at[j]], rows_v, sem).wait()
                        pltpu.sync_copy(
                            rows_v, acc.at[dst_v.at[j]], add=True)
                        return carry

                    lax.fori_loop(0, EB, body, 0)
                    plsc.subcore_barrier()

                    for k in range(ROWS_PER_SUB // 128):
                        pltpu.sync_copy(
                            acc.at[pl.ds(row0 + k * 128, 128)], stage)
                        pltpu.sync_copy(
                            stage, out_hbm.at[pl.ds(row0 + k * 128, 128)])

    return seg


@functools.lru_cache(maxsize=None)
def _get_segsum(num_chunks):
    return _make_segsum(num_chunks)


NOC = D_H // CW


def _dense_layer(agg_chunks, h_chunks, W_rel, Wf, bf, gamma, beta):
    C = len(h_chunks)
    Din = CW * C

    def body(*refs):
        aggs = refs[:C]
        hs = refs[C:2 * C]
        w_rel, wf, bfr, gr, br = refs[2 * C:2 * C + 5]
        outs = refs[2 * C + 5:]
        a = jnp.concatenate([r[...] for r in aggs], axis=1)
        x = jnp.concatenate([r[...] for r in hs], axis=1)
        h = (jnp.dot(a, w_rel[...], preferred_element_type=jnp.float32)
             + jnp.dot(x, wf[...], preferred_element_type=jnp.float32)
             + bfr[...])
        mu = jnp.mean(h, axis=1, keepdims=True)
        hc = h - mu
        var = jnp.mean(hc * hc, axis=1, keepdims=True)
        h = hc * lax.rsqrt(var + EPS) * gr[...] + br[...]
        h = jnp.maximum(h, 0.0)
        for c in range(NOC):
            outs[c][...] = h[:, c * CW:(c + 1) * CW]

    in_specs = (
        [pl.BlockSpec((BLK, CW), lambda i: (i, 0))] * (2 * C)
        + [pl.BlockSpec((Din, D_H), lambda i: (0, 0))] * 2
        + [pl.BlockSpec((1, D_H), lambda i: (0, 0))] * 3
    )
    out_specs = [pl.BlockSpec((BLK, CW), lambda i: (i, 0))] * NOC
    out_shape = [jax.ShapeDtypeStruct((NPAD, CW), jnp.float32)] * NOC
    fn = pl.pallas_call(
        body,
        grid=(NBLK,),
        in_specs=in_specs,
        out_specs=out_specs,
        out_shape=out_shape,
        compiler_params=pltpu.CompilerParams(
            dimension_semantics=("parallel",)),
    )
    return fn(*agg_chunks, *h_chunks, W_rel, Wf,
              bf.reshape(1, D_H), gamma.reshape(1, D_H), beta.reshape(1, D_H))


def _final_layer(agg_chunks, h_chunks, W_rel, Wf, bf, gamma, beta,
                 batch3d, W_final):
    C = len(h_chunks)
    Din = CW * C

    def body(*refs):
        aggs = refs[:C]
        hs = refs[C:2 * C]
        w_rel, wf, bfr, gr, br, batch_r, w_fin = refs[2 * C:2 * C + 7]
        out_r = refs[2 * C + 7]
        sums, counts = refs[2 * C + 8:]

        i = pl.program_id(0)

        a = jnp.concatenate([r[...] for r in aggs], axis=1)
        x = jnp.concatenate([r[...] for r in hs], axis=1)
        h = (jnp.dot(a, w_rel[...], preferred_element_type=jnp.float32)
             + jnp.dot(x, wf[...], preferred_element_type=jnp.float32)
             + bfr[...])
        mu = jnp.mean(h, axis=1, keepdims=True)
        hc = h - mu
        var = jnp.mean(hc * hc, axis=1, keepdims=True)
        h = hc * lax.rsqrt(var + EPS) * gr[...] + br[...]
        h = jnp.maximum(h, 0.0)

        b = batch_r[...].reshape(1, BLK)
        gi = lax.broadcasted_iota(jnp.int32, (G, BLK), 0)
        P = (gi == b).astype(jnp.float32)
        psum = jnp.dot(P, h, preferred_element_type=jnp.float32)
        pcnt = jnp.sum(P, axis=1, keepdims=True)

        @pl.when(i == 0)
        def _():
            sums[...] = jnp.zeros_like(sums)
            counts[...] = jnp.zeros_like(counts)

        sums[...] += psum
        counts[...] += jnp.broadcast_to(pcnt, (G, 128))

        @pl.when(i == NBLK - 1)
        def _():
            pooled = sums[...] / jnp.maximum(counts[...][:, 0:1], 1.0)
            out_r[...] = jnp.dot(pooled, w_fin[...],
                                 preferred_element_type=jnp.float32)

    in_specs = (
        [pl.BlockSpec((BLK, CW), lambda i: (i, 0))] * (2 * C)
        + [pl.BlockSpec((Din, D_H), lambda i: (0, 0))] * 2
        + [pl.BlockSpec((1, D_H), lambda i: (0, 0))] * 3
        + [pl.BlockSpec((1, 1, BLK), lambda i: (i, 0, 0)),
           pl.BlockSpec((D_H, D_OUT), lambda i: (0, 0))]
    )
    fn = pl.pallas_call(
        body,
        grid=(NBLK,),
        in_specs=in_specs,
        out_specs=pl.BlockSpec((G, D_OUT), lambda i: (0, 0)),
        out_shape=jax.ShapeDtypeStruct((G, D_OUT), jnp.float32),
        scratch_shapes=[pltpu.VMEM((G, D_H), jnp.float32),
                        pltpu.VMEM((G, 128), jnp.float32)],
        compiler_params=pltpu.CompilerParams(
            dimension_semantics=("arbitrary",)),
    )
    return fn(*agg_chunks, *h_chunks, W_rel, Wf,
              bf.reshape(1, D_H), gamma.reshape(1, D_H), beta.reshape(1, D_H),
              batch3d, W_final)


@jax.jit
def kernel(x, edge_index, batch,
           W_rel0, b_rel0, W_root0, W_res0, b_res0, gamma0, beta0,
           W_rel1, b_rel1, W_root1, W_res1, b_res1, gamma1, beta1,
           W_rel2, b_rel2, W_root2, W_res2, b_res2, gamma2, beta2,
           W_final):
    src = edge_index[0]
    dst = edge_index[1]
    srcp = jnp.concatenate(
        [src, jnp.zeros((EPAD - E,), jnp.int32)]).reshape(NSUB, EB, 128)
    dstp = jnp.concatenate(
        [dst, jnp.full((EPAD - E,), NPAD - 1, jnp.int32)]).reshape(NSUB, EB, 128)
    zeros = jnp.zeros((128, CW), jnp.float32)

    xp = jnp.pad(x, ((0, NPAD - N), (0, 0)))
    h_chunks = [xp[:, c * CW:(c + 1) * CW] for c in range(D_IN // CW)]
    batch3d = jnp.concatenate(
        [batch, jnp.full((NPAD - N,), G, jnp.int32)]).reshape(NBLK, 1, BLK)

    params = [
        (W_rel0, W_root0 + W_res0, b_rel0 + b_res0, gamma0, beta0),
        (W_rel1, W_root1 + W_res1, b_rel1 + b_res1, gamma1, beta1),
        (W_rel2, W_root2 + W_res2, b_rel2 + b_res2, gamma2, beta2),
    ]

    aggs = _get_segsum(4)(*h_chunks, srcp, dstp, zeros)
    h_chunks = _dense_layer(aggs, h_chunks, *params[0])

    aggs = _get_segsum(8)(*h_chunks, srcp, dstp, zeros)
    h_chunks = _dense_layer(aggs, h_chunks, *params[1])

    aggs = _get_segsum(8)(*h_chunks, srcp, dstp, zeros)
    return _final_layer(aggs, h_chunks, *params[2], batch3d, W_final)


# speedup vs baseline: 2.6489x; 1.0145x over previous
import functools

import jax
import jax.numpy as jnp
from jax import lax
from jax.experimental import pallas as pl
from jax.experimental.pallas import tpu as pltpu
from jax.experimental.pallas import tpu_sc as plsc

N = 10000
E = 160000
D_IN = 256
D_H = 512
D_OUT = 128
G = 64
EPS = 1e-5

NPAD = 10240
BLK = 256
NBLK = NPAD // BLK
NSUB = 16
NCORE = 2
CW = 64
EB = 80
ET = EB * 128
EPAD = NSUB * ET
ROWS_PER_SUB = NPAD // NSUB


def _make_segsum(num_chunks):
    C = num_chunks
    CPC = C // NCORE

    mesh = plsc.VectorSubcoreMesh(core_axis_name="c", subcore_axis_name="s",
                                  num_cores=NCORE, num_subcores=NSUB)
    out_type = [jax.ShapeDtypeStruct((NPAD, CW), jnp.float32)] * C
    scratch = [
        pltpu.VMEM((EB, 128), jnp.int32),
        pltpu.VMEM((EB, 128), jnp.int32),
        pltpu.VMEM((128, CW), jnp.float32),
        pltpu.VMEM((128, CW), jnp.float32),
        pltpu.VMEM((128, CW), jnp.float32),
        pltpu.VMEM((128, CW), jnp.float32),
        pltpu.VMEM((128, CW), jnp.float32),
        pltpu.VMEM((128, CW), jnp.float32),
        pltpu.VMEM_SHARED((NPAD, CW), jnp.float32),
        pltpu.SemaphoreType.DMA,
        pltpu.SemaphoreType.DMA,
    ]

    @functools.partial(pl.kernel, out_type=out_type, mesh=mesh,
                       scratch_types=scratch,
                       compiler_params=pltpu.CompilerParams(
                           use_tc_tiling_on_sc=False))
    def seg(*refs):
        h_refs = refs[:C]
        src_hbm = refs[C]
        dst_hbm = refs[C + 1]
        zeros_hbm = refs[C + 2]
        out_refs = refs[C + 3:C + 3 + C]
        (src_v, dst_v, rA0, rA1, rB0, rB1, zbuf, stage, acc,
         semA, semB) = refs[C + 3 + C:]

        cid = lax.axis_index("c")
        sid = lax.axis_index("s")
        row0 = sid * ROWS_PER_SUB

        pltpu.sync_copy(src_hbm.at[sid], src_v)
        pltpu.sync_copy(dst_hbm.at[sid], dst_v)
        pltpu.sync_copy(zeros_hbm, zbuf)

        for cc in range(NCORE):
            @pl.when(cid == cc)
            def _():
                for q in range(CPC):
                    ch = cc * CPC + q
                    h_hbm = h_refs[ch]
                    out_hbm = out_refs[ch]

                    for k in range(ROWS_PER_SUB // 128):
                        pltpu.sync_copy(
                            zbuf, acc.at[pl.ds(row0 + k * 128, 128)])
                    plsc.subcore_barrier()

                    def gather(j, buf, sem):
                        return pltpu.async_copy(
                            h_hbm.at[src_v.at[j]], buf, sem)

                    def drain(buf, sem):
                        pltpu.make_async_copy(
                            h_hbm.at[pl.ds(0, 128)], buf, sem).wait()

                    def scat(j, buf):
                        pltpu.sync_copy(buf, acc.at[dst_v.at[j]], add=True)

                    gather(0, rA0, semA)
                    gather(1, rA1, semA)

                    def body(t, carry):
                        jA = 4 * t
                        jB = 4 * t + 2
                        gather(jB, rB0, semB)
                        gather(jB + 1, rB1, semB)
                        drain(rA0, semA)
                        drain(rA1, semA)
                        scat(jA, rA0)
                        scat(jA + 1, rA1)

                        @pl.when(t < EB // 4 - 1)
                        def _():
                            gather(jA + 4, rA0, semA)
                            gather(jA + 5, rA1, semA)

                        drain(rB0, semB)
                        drain(rB1, semB)
                        scat(jB, rB0)
                        scat(jB + 1, rB1)
                        return carry

                    lax.fori_loop(0, EB // 4, body, 0)
                    plsc.subcore_barrier()

                    for k in range(ROWS_PER_SUB // 128):
                        pltpu.sync_copy(
                            acc.at[pl.ds(row0 + k * 128, 128)], stage)
                        pltpu.sync_copy(
                            stage, out_hbm.at[pl.ds(row0 + k * 128, 128)])

    return seg


@functools.lru_cache(maxsize=None)
def _get_segsum(num_chunks):
    return _make_segsum(num_chunks)


NOC = D_H // CW


def _dense_layer(agg_chunks, h_chunks, W_rel, Wf, bf, gamma, beta):
    C = len(h_chunks)
    Din = CW * C

    def body(*refs):
        aggs = refs[:C]
        hs = refs[C:2 * C]
        w_rel, wf, bfr, gr, br = refs[2 * C:2 * C + 5]
        outs = refs[2 * C + 5:]
        a = jnp.concatenate([r[...] for r in aggs], axis=1)
        x = jnp.concatenate([r[...] for r in hs], axis=1)
        h = (jnp.dot(a, w_rel[...], preferred_element_type=jnp.float32)
             + jnp.dot(x, wf[...], preferred_element_type=jnp.float32)
             + bfr[...])
        mu = jnp.mean(h, axis=1, keepdims=True)
        hc = h - mu
        var = jnp.mean(hc * hc, axis=1, keepdims=True)
        h = hc * lax.rsqrt(var + EPS) * gr[...] + br[...]
        h = jnp.maximum(h, 0.0)
        for c in range(NOC):
            outs[c][...] = h[:, c * CW:(c + 1) * CW]

    in_specs = (
        [pl.BlockSpec((BLK, CW), lambda i: (i, 0))] * (2 * C)
        + [pl.BlockSpec((Din, D_H), lambda i: (0, 0))] * 2
        + [pl.BlockSpec((1, D_H), lambda i: (0, 0))] * 3
    )
    out_specs = [pl.BlockSpec((BLK, CW), lambda i: (i, 0))] * NOC
    out_shape = [jax.ShapeDtypeStruct((NPAD, CW), jnp.float32)] * NOC
    fn = pl.pallas_call(
        body,
        grid=(NBLK,),
        in_specs=in_specs,
        out_specs=out_specs,
        out_shape=out_shape,
        compiler_params=pltpu.CompilerParams(
            dimension_semantics=("parallel",)),
    )
    return fn(*agg_chunks, *h_chunks, W_rel, Wf,
              bf.reshape(1, D_H), gamma.reshape(1, D_H), beta.reshape(1, D_H))


def _final_layer(agg_chunks, h_chunks, W_rel, Wf, bf, gamma, beta,
                 batch3d, W_final):
    C = len(h_chunks)
    Din = CW * C

    def body(*refs):
        aggs = refs[:C]
        hs = refs[C:2 * C]
        w_rel, wf, bfr, gr, br, batch_r, w_fin = refs[2 * C:2 * C + 7]
        out_r = refs[2 * C + 7]
        sums, counts = refs[2 * C + 8:]

        i = pl.program_id(0)

        a = jnp.concatenate([r[...] for r in aggs], axis=1)
        x = jnp.concatenate([r[...] for r in hs], axis=1)
        h = (jnp.dot(a, w_rel[...], preferred_element_type=jnp.float32)
             + jnp.dot(x, wf[...], preferred_element_type=jnp.float32)
             + bfr[...])
        mu = jnp.mean(h, axis=1, keepdims=True)
        hc = h - mu
        var = jnp.mean(hc * hc, axis=1, keepdims=True)
        h = hc * lax.rsqrt(var + EPS) * gr[...] + br[...]
        h = jnp.maximum(h, 0.0)

        b = batch_r[...].reshape(1, BLK)
        gi = lax.broadcasted_iota(jnp.int32, (G, BLK), 0)
        P = (gi == b).astype(jnp.float32)
        psum = jnp.dot(P, h, preferred_element_type=jnp.float32)
        pcnt = jnp.sum(P, axis=1, keepdims=True)

        @pl.when(i == 0)
        def _():
            sums[...] = jnp.zeros_like(sums)
            counts[...] = jnp.zeros_like(counts)

        sums[...] += psum
        counts[...] += jnp.broadcast_to(pcnt, (G, 128))

        @pl.when(i == NBLK - 1)
        def _():
            pooled = sums[...] / jnp.maximum(counts[...][:, 0:1], 1.0)
            out_r[...] = jnp.dot(pooled, w_fin[...],
                                 preferred_element_type=jnp.float32)

    in_specs = (
        [pl.BlockSpec((BLK, CW), lambda i: (i, 0))] * (2 * C)
        + [pl.BlockSpec((Din, D_H), lambda i: (0, 0))] * 2
        + [pl.BlockSpec((1, D_H), lambda i: (0, 0))] * 3
        + [pl.BlockSpec((1, 1, BLK), lambda i: (i, 0, 0)),
           pl.BlockSpec((D_H, D_OUT), lambda i: (0, 0))]
    )
    fn = pl.pallas_call(
        body,
        grid=(NBLK,),
        in_specs=in_specs,
        out_specs=pl.BlockSpec((G, D_OUT), lambda i: (0, 0)),
        out_shape=jax.ShapeDtypeStruct((G, D_OUT), jnp.float32),
        scratch_shapes=[pltpu.VMEM((G, D_H), jnp.float32),
                        pltpu.VMEM((G, 128), jnp.float32)],
        compiler_params=pltpu.CompilerParams(
            dimension_semantics=("arbitrary",)),
    )
    return fn(*agg_chunks, *h_chunks, W_rel, Wf,
              bf.reshape(1, D_H), gamma.reshape(1, D_H), beta.reshape(1, D_H),
              batch3d, W_final)


@jax.jit
def kernel(x, edge_index, batch,
           W_rel0, b_rel0, W_root0, W_res0, b_res0, gamma0, beta0,
           W_rel1, b_rel1, W_root1, W_res1, b_res1, gamma1, beta1,
           W_rel2, b_rel2, W_root2, W_res2, b_res2, gamma2, beta2,
           W_final):
    src = edge_index[0]
    dst = edge_index[1]
    srcp = jnp.concatenate(
        [src, jnp.zeros((EPAD - E,), jnp.int32)]).reshape(NSUB, EB, 128)
    dstp = jnp.concatenate(
        [dst, jnp.full((EPAD - E,), NPAD - 1, jnp.int32)]).reshape(NSUB, EB, 128)
    zeros = jnp.zeros((128, CW), jnp.float32)

    xp = jnp.pad(x, ((0, NPAD - N), (0, 0)))
    h_chunks = [xp[:, c * CW:(c + 1) * CW] for c in range(D_IN // CW)]
    batch3d = jnp.concatenate(
        [batch, jnp.full((NPAD - N,), G, jnp.int32)]).reshape(NBLK, 1, BLK)

    params = [
        (W_rel0, W_root0 + W_res0, b_rel0 + b_res0, gamma0, beta0),
        (W_rel1, W_root1 + W_res1, b_rel1 + b_res1, gamma1, beta1),
        (W_rel2, W_root2 + W_res2, b_rel2 + b_res2, gamma2, beta2),
    ]

    aggs = _get_segsum(4)(*h_chunks, srcp, dstp, zeros)
    h_chunks = _dense_layer(aggs, h_chunks, *params[0])

    aggs = _get_segsum(8)(*h_chunks, srcp, dstp, zeros)
    h_chunks = _dense_layer(aggs, h_chunks, *params[1])

    aggs = _get_segsum(8)(*h_chunks, srcp, dstp, zeros)
    return _final_layer(aggs, h_chunks, *params[2], batch3d, W_final)


# speedup vs baseline: 4.0898x; 1.5440x over previous
import functools

import jax
import jax.numpy as jnp
from jax import lax
from jax.experimental import pallas as pl
from jax.experimental.pallas import tpu as pltpu
from jax.experimental.pallas import tpu_sc as plsc

N = 10000
E = 160000
D_IN = 256
D_H = 512
D_OUT = 128
G = 64
EPS = 1e-5

NPAD = 10240
BLK = 256
NBLK = NPAD // BLK
NSUB = 16
NCORE = 2
EB = 80
ET = EB * 128
EPAD = NSUB * ET
ROWS_PER_SUB = NPAD // NSUB


def _make_segsum(cw):
    C = NCORE

    mesh = plsc.VectorSubcoreMesh(core_axis_name="c", subcore_axis_name="s",
                                  num_cores=NCORE, num_subcores=NSUB)
    out_type = [jax.ShapeDtypeStruct((NPAD, cw), jnp.bfloat16)] * C
    scratch = [
        pltpu.VMEM((EB, 128), jnp.int32),
        pltpu.VMEM((EB, 128), jnp.int32),
        pltpu.VMEM((128, cw), jnp.bfloat16),
        pltpu.VMEM((64, cw), jnp.bfloat16),
        pltpu.VMEM_SHARED((NPAD, cw), jnp.bfloat16),
        pltpu.SemaphoreType.DMA,
    ]

    @functools.partial(pl.kernel, out_type=out_type, mesh=mesh,
                       scratch_types=scratch,
                       compiler_params=pltpu.CompilerParams(
                           use_tc_tiling_on_sc=False))
    def seg(*refs):
        h_refs = refs[:C]
        src_hbm = refs[C]
        dst_hbm = refs[C + 1]
        zeros_hbm = refs[C + 2]
        out_refs = refs[C + 3:C + 3 + C]
        src_v, dst_v, rows_v, zbuf, acc, sem = refs[C + 3 + C:]

        cid = lax.axis_index("c")
        sid = lax.axis_index("s")
        row0 = sid * ROWS_PER_SUB

        pltpu.sync_copy(src_hbm.at[sid], src_v)
        pltpu.sync_copy(dst_hbm.at[sid], dst_v)
        pltpu.sync_copy(zeros_hbm, zbuf)

        for cc in range(NCORE):
            @pl.when(cid == cc)
            def _():
                h_hbm = h_refs[cc]
                out_hbm = out_refs[cc]

                for k in range(ROWS_PER_SUB // 64):
                    pltpu.sync_copy(
                        zbuf, acc.at[pl.ds(row0 + k * 64, 64)])
                plsc.subcore_barrier()

                def body(j, carry):
                    pltpu.async_copy(
                        h_hbm.at[src_v.at[j]], rows_v, sem).wait()
                    pltpu.sync_copy(
                        rows_v, acc.at[dst_v.at[j]], add=True)
                    return carry

                lax.fori_loop(0, EB, body, 0)
                plsc.subcore_barrier()

                for k in range(ROWS_PER_SUB // 128):
                    pltpu.sync_copy(
                        acc.at[pl.ds(row0 + k * 128, 128)], rows_v)
                    pltpu.sync_copy(
                        rows_v, out_hbm.at[pl.ds(row0 + k * 128, 128)])

    return seg


@functools.lru_cache(maxsize=None)
def _get_segsum(cw):
    return _make_segsum(cw)


OCW = D_H // NCORE


def _dense_layer(agg_chunks, h_chunks, W_rel, Wf, bf, gamma, beta):
    cw_in = agg_chunks[0].shape[1]
    Din = cw_in * NCORE

    def body(*refs):
        aggs = refs[:NCORE]
        hs = refs[NCORE:2 * NCORE]
        w_rel, wf, bfr, gr, br = refs[2 * NCORE:2 * NCORE + 5]
        outs = refs[2 * NCORE + 5:]
        a = jnp.concatenate(
            [r[...] for r in aggs], axis=1).astype(jnp.float32)
        x = jnp.concatenate(
            [r[...] for r in hs], axis=1).astype(jnp.float32)
        h = (jnp.dot(a, w_rel[...], preferred_element_type=jnp.float32)
             + jnp.dot(x, wf[...], preferred_element_type=jnp.float32)
             + bfr[...])
        mu = jnp.mean(h, axis=1, keepdims=True)
        hc = h - mu
        var = jnp.mean(hc * hc, axis=1, keepdims=True)
        h = hc * lax.rsqrt(var + EPS) * gr[...] + br[...]
        h = jnp.maximum(h, 0.0).astype(jnp.bfloat16)
        for c in range(NCORE):
            outs[c][...] = h[:, c * OCW:(c + 1) * OCW]

    in_specs = (
        [pl.BlockSpec((BLK, cw_in), lambda i: (i, 0))] * (2 * NCORE)
        + [pl.BlockSpec((Din, D_H), lambda i: (0, 0))] * 2
        + [pl.BlockSpec((1, D_H), lambda i: (0, 0))] * 3
    )
    out_specs = [pl.BlockSpec((BLK, OCW), lambda i: (i, 0))] * NCORE
    out_shape = [jax.ShapeDtypeStruct((NPAD, OCW), jnp.bfloat16)] * NCORE
    fn = pl.pallas_call(
        body,
        grid=(NBLK,),
        in_specs=in_specs,
        out_specs=out_specs,
        out_shape=out_shape,
        compiler_params=pltpu.CompilerParams(
            dimension_semantics=("parallel",)),
    )
    return fn(*agg_chunks, *h_chunks, W_rel, Wf,
              bf.reshape(1, D_H), gamma.reshape(1, D_H), beta.reshape(1, D_H))


def _final_layer(agg_chunks, h_chunks, W_rel, Wf, bf, gamma, beta,
                 batch3d, W_final):
    cw_in = agg_chunks[0].shape[1]
    Din = cw_in * NCORE

    def body(*refs):
        aggs = refs[:NCORE]
        hs = refs[NCORE:2 * NCORE]
        w_rel, wf, bfr, gr, br, batch_r, w_fin = refs[2 * NCORE:2 * NCORE + 7]
        out_r = refs[2 * NCORE + 7]
        sums, counts = refs[2 * NCORE + 8:]

        i = pl.program_id(0)

        a = jnp.concatenate(
            [r[...] for r in aggs], axis=1).astype(jnp.float32)
        x = jnp.concatenate(
            [r[...] for r in hs], axis=1).astype(jnp.float32)
        h = (jnp.dot(a, w_rel[...], preferred_element_type=jnp.float32)
             + jnp.dot(x, wf[...], preferred_element_type=jnp.float32)
             + bfr[...])
        mu = jnp.mean(h, axis=1, keepdims=True)
        hc = h - mu
        var = jnp.mean(hc * hc, axis=1, keepdims=True)
        h = hc * lax.rsqrt(var + EPS) * gr[...] + br[...]
        h = jnp.maximum(h, 0.0)

        b = batch_r[...].reshape(1, BLK)
        gi = lax.broadcasted_iota(jnp.int32, (G, BLK), 0)
        P = (gi == b).astype(jnp.float32)
        psum = jnp.dot(P, h, preferred_element_type=jnp.float32)
        pcnt = jnp.sum(P, axis=1, keepdims=True)

        @pl.when(i == 0)
        def _():
            sums[...] = jnp.zeros_like(sums)
            counts[...] = jnp.zeros_like(counts)

        sums[...] += psum
        counts[...] += jnp.broadcast_to(pcnt, (G, 128))

        @pl.when(i == NBLK - 1)
        def _():
            pooled = sums[...] / jnp.maximum(counts[...][:, 0:1], 1.0)
            out_r[...] = jnp.dot(pooled, w_fin[...],
                                 preferred_element_type=jnp.float32)

    in_specs = (
        [pl.BlockSpec((BLK, cw_in), lambda i: (i, 0))] * (2 * NCORE)
        + [pl.BlockSpec((Din, D_H), lambda i: (0, 0))] * 2
        + [pl.BlockSpec((1, D_H), lambda i: (0, 0))] * 3
        + [pl.BlockSpec((1, 1, BLK), lambda i: (i, 0, 0)),
           pl.BlockSpec((D_H, D_OUT), lambda i: (0, 0))]
    )
    fn = pl.pallas_call(
        body,
        grid=(NBLK,),
        in_specs=in_specs,
        out_specs=pl.BlockSpec((G, D_OUT), lambda i: (0, 0)),
        out_shape=jax.ShapeDtypeStruct((G, D_OUT), jnp.float32),
        scratch_shapes=[pltpu.VMEM((G, D_H), jnp.float32),
                        pltpu.VMEM((G, 128), jnp.float32)],
        compiler_params=pltpu.CompilerParams(
            dimension_semantics=("arbitrary",)),
    )
    return fn(*agg_chunks, *h_chunks, W_rel, Wf,
              bf.reshape(1, D_H), gamma.reshape(1, D_H), beta.reshape(1, D_H),
              batch3d, W_final)


@jax.jit
def kernel(x, edge_index, batch,
           W_rel0, b_rel0, W_root0, W_res0, b_res0, gamma0, beta0,
           W_rel1, b_rel1, W_root1, W_res1, b_res1, gamma1, beta1,
           W_rel2, b_rel2, W_root2, W_res2, b_res2, gamma2, beta2,
           W_final):
    src = edge_index[0]
    dst = edge_index[1]
    srcp = jnp.concatenate(
        [src, jnp.zeros((EPAD - E,), jnp.int32)]).reshape(NSUB, EB, 128)
    dstp = jnp.concatenate(
        [dst, jnp.full((EPAD - E,), NPAD - 1, jnp.int32)]).reshape(NSUB, EB, 128)
    zeros128 = jnp.zeros((64, 128), jnp.bfloat16)
    zeros256 = jnp.zeros((64, OCW), jnp.bfloat16)

    xp = jnp.pad(x, ((0, NPAD - N), (0, 0))).astype(jnp.bfloat16)
    h_chunks = [xp[:, :128], xp[:, 128:]]
    batch3d = jnp.concatenate(
        [batch, jnp.full((NPAD - N,), G, jnp.int32)]).reshape(NBLK, 1, BLK)

    params = [
        (W_rel0, W_root0 + W_res0, b_rel0 + b_res0, gamma0, beta0),
        (W_rel1, W_root1 + W_res1, b_rel1 + b_res1, gamma1, beta1),
        (W_rel2, W_root2 + W_res2, b_rel2 + b_res2, gamma2, beta2),
    ]

    aggs = _get_segsum(128)(*h_chunks, srcp, dstp, zeros128)
    h_chunks = _dense_layer(aggs, h_chunks, *params[0])

    aggs = _get_segsum(OCW)(*h_chunks, srcp, dstp, zeros256)
    h_chunks = _dense_layer(aggs, h_chunks, *params[1])

    aggs = _get_segsum(OCW)(*h_chunks, srcp, dstp, zeros256)
    return _final_layer(aggs, h_chunks, *params[2], batch3d, W_final)
